# baseline (speedup 1.0000x reference)
import functools

import jax
import jax.numpy as jnp
from jax import lax
from jax.experimental import pallas as pl
from jax.experimental.pallas import tpu as pltpu
from jax.experimental.pallas import tpu_sc as plsc

BATCH = 16384
VEC = 64
NOUT = 30

_NC = 2
_NS = 16
_NW = _NC * _NS
_BPW = BATCH // _NW
_CH = 128


def _sc_gather(batter_idx, pitcher_idx, batter_table, pitcher_table):
    mesh = plsc.VectorSubcoreMesh(core_axis_name="c", subcore_axis_name="s")

    @functools.partial(
        pl.kernel,
        mesh=mesh,
        out_type=[
            jax.ShapeDtypeStruct((BATCH, VEC), jnp.float32),
            jax.ShapeDtypeStruct((BATCH, VEC), jnp.float32),
        ],
        scratch_types=[
            pltpu.VMEM((_BPW,), jnp.int32),
            pltpu.VMEM((_BPW,), jnp.int32),
            pltpu.VMEM((_BPW, VEC), jnp.float32),
            pltpu.VMEM((_BPW, VEC), jnp.float32),
            pltpu.SemaphoreType.DMA,
        ],
        compiler_params=pltpu.CompilerParams(use_tc_tiling_on_sc=False),
    )
    def k(bidx_hbm, pidx_hbm, btab_hbm, ptab_hbm, bout_hbm, pout_hbm,
          bidx_v, pidx_v, brows_v, prows_v, sem):
        wid = lax.axis_index("s") * _NC + lax.axis_index("c")
        base = wid * _BPW
        pltpu.sync_copy(bidx_hbm.at[pl.ds(base, _BPW)], bidx_v)
        pltpu.sync_copy(pidx_hbm.at[pl.ds(base, _BPW)], pidx_v)
        copies = []
        for c in range(_BPW // _CH):
            sl = pl.ds(c * _CH, _CH)
            copies.append(
                pltpu.async_copy(btab_hbm.at[bidx_v.at[sl]], brows_v.at[sl], sem))
            copies.append(
                pltpu.async_copy(ptab_hbm.at[pidx_v.at[sl]], prows_v.at[sl], sem))
        for cp in copies:
            cp.wait()
        pltpu.sync_copy(brows_v, bout_hbm.at[pl.ds(base, _BPW)])
        pltpu.sync_copy(prows_v, pout_hbm.at[pl.ds(base, _BPW)])

    return k(batter_idx, pitcher_idx, batter_table, pitcher_table)


def _tc_body(bemb_ref, pemb_ref, w1_ref, w2_ref, bias_ref, out_ref):
    sb = jax.nn.sigmoid(bemb_ref[...])
    sp = jax.nn.sigmoid(pemb_ref[...])
    acc = jnp.dot(sb, w1_ref[...], preferred_element_type=jnp.float32)
    acc += jnp.dot(sp, w2_ref[...], preferred_element_type=jnp.float32)
    out_ref[...] = acc + bias_ref[...]


def _tc_dense(bemb, pemb, fc_w, fc_b):
    w1 = fc_w[:, :VEC].T
    w2 = fc_w[:, VEC:].T
    bias = fc_b.reshape(1, NOUT)
    blk = 2048
    grid = (BATCH // blk,)
    return pl.pallas_call(
        _tc_body,
        grid=grid,
        in_specs=[
            pl.BlockSpec((blk, VEC), lambda i: (i, 0)),
            pl.BlockSpec((blk, VEC), lambda i: (i, 0)),
            pl.BlockSpec((VEC, NOUT), lambda i: (0, 0)),
            pl.BlockSpec((VEC, NOUT), lambda i: (0, 0)),
            pl.BlockSpec((1, NOUT), lambda i: (0, 0)),
        ],
        out_specs=pl.BlockSpec((blk, NOUT), lambda i: (i, 0)),
        out_shape=jax.ShapeDtypeStruct((BATCH, NOUT), jnp.float32),
    )(bemb, pemb, w1, w2, bias)


@jax.jit
def kernel(batter_idx, pitcher_idx, batter_table, pitcher_table, fc_w, fc_b):
    bemb, pemb = _sc_gather(batter_idx, pitcher_idx, batter_table, pitcher_table)
    return _tc_dense(bemb, pemb, fc_w, fc_b)


# speedup vs baseline: 1.5878x; 1.5878x over previous
import functools

import jax
import jax.numpy as jnp
from jax import lax
from jax.experimental import pallas as pl
from jax.experimental.pallas import tpu as pltpu
from jax.experimental.pallas import tpu_sc as plsc

BATCH = 16384
VEC = 64
NOUT = 30
NB = 1000000
NP = 100000
HB = 524288
HP = 65536

_NC = 2
_NS = 16
_NW = _NC * _NS
_BPW = BATCH // _NW
_CH = 128


def _pair_body(lo_ref, hi_ref, out_ref):
    out_ref[...] = jnp.concatenate([lo_ref[...].T, hi_ref[...].T], axis=1)


def _tc_pair(tab_t, half):
    blk = 2048
    grid = (half // blk,)
    off = half // blk
    last = (tab_t.shape[1] + blk - 1) // blk - 1
    return pl.pallas_call(
        _pair_body,
        grid=grid,
        in_specs=[
            pl.BlockSpec((VEC, blk), lambda i: (0, i)),
            pl.BlockSpec((VEC, blk), lambda i: (0, jnp.minimum(i + off, last))),
        ],
        out_specs=pl.BlockSpec((blk, 2 * VEC), lambda i: (i, 0)),
        out_shape=jax.ShapeDtypeStruct((half, 2 * VEC), jnp.float32),
    )(tab_t, tab_t)


def _sc_gather(batter_idx, pitcher_idx, bpair, ppair):
    mesh = plsc.VectorSubcoreMesh(core_axis_name="c", subcore_axis_name="s")

    @functools.partial(
        pl.kernel,
        mesh=mesh,
        out_type=[
            jax.ShapeDtypeStruct((BATCH, 2 * VEC), jnp.float32),
            jax.ShapeDtypeStruct((BATCH, 2 * VEC), jnp.float32),
        ],
        scratch_types=[
            pltpu.VMEM((_BPW,), jnp.int32),
            pltpu.VMEM((_BPW,), jnp.int32),
            pltpu.VMEM((_BPW, 2 * VEC), jnp.float32),
            pltpu.SemaphoreType.DMA,
        ],
    )
    def k(bidx_hbm, pidx_hbm, btab_hbm, ptab_hbm, bout_hbm, pout_hbm,
          bidx_v, pidx_v, rows_v, sem):
        wid = lax.axis_index("s") * _NC + lax.axis_index("c")
        base = wid * _BPW
        pltpu.sync_copy(bidx_hbm.at[pl.ds(base, _BPW)], bidx_v)
        pltpu.sync_copy(pidx_hbm.at[pl.ds(base, _BPW)], pidx_v)
        for g in range(_BPW // 16):
            sl = pl.ds(g * 16, 16)
            bi = bidx_v[sl]
            bidx_v[sl] = jnp.where(bi >= HB, bi - HB, bi)
            pi = pidx_v[sl]
            pidx_v[sl] = jnp.where(pi >= HP, pi - HP, pi)
        for idx_v, tab_hbm, out_hbm in (
            (bidx_v, btab_hbm, bout_hbm),
            (pidx_v, ptab_hbm, pout_hbm),
        ):
            copies = []
            for c in range(_BPW // _CH):
                sl = pl.ds(c * _CH, _CH)
                copies.append(
                    pltpu.async_copy(tab_hbm.at[idx_v.at[sl]], rows_v.at[sl], sem))
            for cp in copies:
                cp.wait()
            pltpu.sync_copy(rows_v, out_hbm.at[pl.ds(base, _BPW)])

    return k(batter_idx, pitcher_idx, bpair, ppair)


def _tc_body(bsel_ref, psel_ref, bpar_ref, ppar_ref, w1_ref, w2_ref, bias_ref,
             out_ref):
    bsel = bsel_ref[...]
    psel = psel_ref[...]
    bemb = jnp.where(bpar_ref[...] > 0.5, bsel[:, VEC:], bsel[:, :VEC])
    pemb = jnp.where(ppar_ref[...] > 0.5, psel[:, VEC:], psel[:, :VEC])
    sb = jax.nn.sigmoid(bemb)
    sp = jax.nn.sigmoid(pemb)
    acc = jnp.dot(sb, w1_ref[...], preferred_element_type=jnp.float32)
    acc += jnp.dot(sp, w2_ref[...], preferred_element_type=jnp.float32)
    out_ref[...] = acc + bias_ref[...]


def _tc_dense(bsel, psel, bpar, ppar, fc_w, fc_b):
    w1 = fc_w[:, :VEC].T
    w2 = fc_w[:, VEC:].T
    bias = fc_b.reshape(1, NOUT)
    blk = 2048
    grid = (BATCH // blk,)
    return pl.pallas_call(
        _tc_body,
        grid=grid,
        in_specs=[
            pl.BlockSpec((blk, 2 * VEC), lambda i: (i, 0)),
            pl.BlockSpec((blk, 2 * VEC), lambda i: (i, 0)),
            pl.BlockSpec((blk, 1), lambda i: (i, 0)),
            pl.BlockSpec((blk, 1), lambda i: (i, 0)),
            pl.BlockSpec((VEC, NOUT), lambda i: (0, 0)),
            pl.BlockSpec((VEC, NOUT), lambda i: (0, 0)),
            pl.BlockSpec((1, NOUT), lambda i: (0, 0)),
        ],
        out_specs=pl.BlockSpec((blk, NOUT), lambda i: (i, 0)),
        out_shape=jax.ShapeDtypeStruct((BATCH, NOUT), jnp.float32),
    )(bsel, psel, bpar, ppar, w1, w2, bias)


@jax.jit
def kernel(batter_idx, pitcher_idx, batter_table, pitcher_table, fc_w, fc_b):
    bpair = _tc_pair(batter_table.T, HB)
    ppair = _tc_pair(pitcher_table.T, HP)
    bsel, psel = _sc_gather(batter_idx, pitcher_idx, bpair, ppair)
    bpar = (batter_idx >= HB).astype(jnp.float32).reshape(-1, 1)
    ppar = (pitcher_idx >= HP).astype(jnp.float32).reshape(-1, 1)
    return _tc_dense(bsel, psel, bpar, ppar, fc_w, fc_b)
